# Initial kernel scaffold; baseline (speedup 1.0000x reference)
import jax
import jax.numpy as jnp
from jax.experimental import pallas as pl


def kernel(x, edge_index, Wl0, bl0, Wr0, br0, att0, cb0, Wl1, bl1, Wr1, br1, att1, cb1, Wl2, bl2, Wr2, br2, att2, cb2, fcW1, fcb1, fcW2, fcb2, fcW3, fcb3):
    raise NotImplementedError("write your pallas kernel here")


# speedup vs baseline: 8.8665x; 8.8665x over previous
import jax
import jax.numpy as jnp
from jax import lax
from jax.experimental import pallas as pl
from jax.experimental.pallas import tpu as pltpu

_N = 256
_E = 16384
_DH = 512
_EC = 2048
_DB = 16
_BK = 8192
_NBK = (_N * _DH) // _BK

_f32 = jnp.float32


def _gat3_body(src_ref, dst_ref, x_ref,
               Wl0, bl0, Wr0, br0, att0, cb0,
               Wl1, bl1, Wr1, br1, att1, cb1,
               Wl2, bl2, Wr2, br2, att2, cb2,
               h_out):
    def cstep(i, C):
        srow = src_ref[pl.ds(i, 1), :]
        dcol = dst_ref[pl.ds(i * _EC, _EC), :]
        ohsT = (lax.broadcasted_iota(jnp.int32, (_N, _EC), 0) == srow
                ).astype(_f32)
        ohd = (lax.broadcasted_iota(jnp.int32, (_EC, _N), 1) == dcol
               ).astype(_f32)
        return C + jnp.dot(ohsT, ohd, preferred_element_type=_f32)

    eye = (lax.broadcasted_iota(jnp.int32, (_N, _N), 0)
           == lax.broadcasted_iota(jnp.int32, (_N, _N), 1)).astype(_f32)
    C = lax.fori_loop(0, _E // _EC, cstep, eye)
    negmask = jnp.where(C > 0.0, 0.0, -3e38)

    def layer(h, Wl, bl, Wr, br, att, cb):
        xl = jnp.dot(h, Wl[:], preferred_element_type=_f32) + bl[:]
        xr = jnp.dot(h, Wr[:], preferred_element_type=_f32) + br[:]
        attv = att[:]

        blocks = []
        for i in range(_N // _DB):
            xrb = xr[i * _DB:(i + 1) * _DB, :]
            z = xl[:, None, :] + xrb[None, :, :]
            m = jnp.where(z >= 0.0, z, 0.2 * z)
            blocks.append(jnp.sum(m * attv[None, :, :], axis=-1))
        alpha = jnp.concatenate(blocks, axis=1)
        amax = jnp.max(alpha + negmask, axis=0, keepdims=True)
        ex = C * jnp.exp(jnp.minimum(alpha - amax, 0.0))
        denom = jnp.sum(ex, axis=0, keepdims=True)
        A = ex / denom
        out = lax.dot_general(A, xl, (((0,), (0,)), ((), ())),
                              preferred_element_type=_f32)
        return jnp.tanh(out + cb[:])

    h = layer(x_ref[:], Wl0, bl0, Wr0, br0, att0, cb0)
    h = layer(h, Wl1, bl1, Wr1, br1, att1, cb1)
    h = layer(h, Wl2, bl2, Wr2, br2, att2, cb2)
    h_out[:] = h


def _fc_body(hf_ref, W1_ref, b1_ref, W2_ref, b2_ref, W3_ref, b3_ref,
             out_ref, acc_ref):
    i = pl.program_id(0)
    part = jnp.dot(hf_ref[:], W1_ref[:], preferred_element_type=_f32)

    @pl.when(i == 0)
    def _():
        acc_ref[:] = part

    @pl.when(i > 0)
    def _():
        acc_ref[:] = acc_ref[:] + part

    @pl.when(i == _NBK - 1)
    def _():
        z1 = acc_ref[:] + b1_ref[:]
        a1 = jnp.where(z1 >= 0.0, z1, 0.01 * z1)
        z2 = jnp.dot(a1, W2_ref[:], preferred_element_type=_f32) + b2_ref[:]
        a2 = jnp.where(z2 >= 0.0, z2, 0.01 * z2)
        out_ref[:] = jnp.dot(a2, W3_ref[:], preferred_element_type=_f32) \
            + b3_ref[:]


def kernel(x, edge_index, Wl0, bl0, Wr0, br0, att0, cb0,
           Wl1, bl1, Wr1, br1, att1, cb1,
           Wl2, bl2, Wr2, br2, att2, cb2,
           fcW1, fcb1, fcW2, fcb2, fcW3, fcb3):
    src2d = edge_index[0].reshape(_E // _EC, _EC)
    dcol = edge_index[1].reshape(_E, 1)
    r = lambda v: v.reshape(1, -1)

    h = pl.pallas_call(
        _gat3_body,
        out_shape=jax.ShapeDtypeStruct((_N, _DH), _f32),
    )(src2d, dcol, x,
      Wl0, r(bl0), Wr0, r(br0), r(att0), r(cb0),
      Wl1, r(bl1), Wr1, r(br1), r(att1), r(cb1),
      Wl2, r(bl2), Wr2, r(br2), r(att2), r(cb2))

    hf = h.reshape(1, _N * _DH)
    out = pl.pallas_call(
        _fc_body,
        grid=(_NBK,),
        in_specs=[
            pl.BlockSpec((1, _BK), lambda i: (0, i)),
            pl.BlockSpec((_BK, _DH), lambda i: (i, 0)),
            pl.BlockSpec((1, _DH), lambda i: (0, 0)),
            pl.BlockSpec((_DH, _DH), lambda i: (0, 0)),
            pl.BlockSpec((1, _DH), lambda i: (0, 0)),
            pl.BlockSpec((_DH, 1), lambda i: (0, 0)),
            pl.BlockSpec((1, 1), lambda i: (0, 0)),
        ],
        out_specs=pl.BlockSpec((1, 1), lambda i: (0, 0)),
        out_shape=jax.ShapeDtypeStruct((1, 1), _f32),
        scratch_shapes=[pltpu.VMEM((1, _DH), _f32)],
    )(hf, fcW1, r(fcb1), fcW2, r(fcb2), fcW3, fcb3.reshape(1, 1))
    return out.reshape(1)
